# baseline (speedup 1.0000x reference)
import functools

import jax
import jax.numpy as jnp
from jax import lax
from jax.experimental import pallas as pl
from jax.experimental.pallas import tpu as pltpu
from jax.experimental.pallas import tpu_sc as plsc
import numpy as np

_SHAPE = (4, 1, 96, 128, 128)
_N = int(np.prod(_SHAPE))
_COLS = 128
_ROWS = _N // _COLS

_FACTOR = 1.0
_GAMA = 1.5
_EPS = 1e-08
_LN2 = 0.6931471805599453
_LOG2E = 1.4426950408889634

_SC_WORKERS = 32
_SC_CHUNK = 16384
_SC_K = 1
_N_SC = _SC_WORKERS * _SC_K * _SC_CHUNK
_N_TC = _N - _N_SC
_TC_ROWS = _N_TC // _COLS
_BLK_ROWS = 5632
_GRID = _TC_ROWS // _BLK_ROWS
assert _TC_ROWS % _BLK_ROWS == 0

_CHUNK = 64
_NCHUNK = _BLK_ROWS // _CHUNK

_P0 = 1.4426688879949858
_P1 = -0.7201758374630753
_P2 = 0.46802731058715463
_P3 = -0.30104668882778535
_P4 = 0.14468385867161748
_P5 = -0.03417637074728508


def _tc_body(x_ref, y_ref, out_ref, acc_ref):
    i = pl.program_id(0)

    def _step(k, carry):
        sA, sC, sM = carry
        xb = x_ref[pl.ds(k * _CHUNK, _CHUNK), :]
        yb = y_ref[pl.ds(k * _CHUNK, _CHUNK), :]
        yf = yb.astype(jnp.float32)
        z = xb * (2.0 * yf - 1.0)
        a = jnp.minimum(z * (-_LOG2E), 126.0)
        u = jnp.exp2(a)
        d = 1.0 + u
        L = jnp.log2(d)
        c = jnp.exp2(1.5 * (a - L)) * L
        return (sA + yf * c, sC + c, sM + yf)

    z0 = jnp.zeros((_CHUNK, _COLS), jnp.float32)
    carry = (z0, z0, z0)
    for k in range(_NCHUNK):
        carry = _step(k, carry)
    sA, sC, sM = carry

    @pl.when(i == 0)
    def _init():
        acc_ref[...] = jnp.zeros_like(acc_ref)

    acc_ref[0, :] += jnp.sum(sA, axis=0)
    acc_ref[1, :] += jnp.sum(sC, axis=0)
    acc_ref[2, :] += jnp.sum(sM, axis=0)

    @pl.when(i == _GRID - 1)
    def _fin():
        out_ref[0] = jnp.sum(acc_ref[0, :]) * _LN2
        out_ref[1] = jnp.sum(acc_ref[1, :]) * _LN2
        out_ref[2] = jnp.sum(acc_ref[2, :])


def _tc_call(x2, y2):
    return pl.pallas_call(
        _tc_body,
        grid=(_GRID,),
        in_specs=[
            pl.BlockSpec((_BLK_ROWS, _COLS), lambda i: (i, 0)),
            pl.BlockSpec((_BLK_ROWS, _COLS), lambda i: (i, 0)),
        ],
        out_specs=pl.BlockSpec(memory_space=pltpu.SMEM),
        out_shape=jax.ShapeDtypeStruct((3,), jnp.float32),
        scratch_shapes=[pltpu.VMEM((3, _COLS), jnp.float32)],
    )(x2, y2)


def _sc_chunk_sums(xbuf, ybuf, carry_in):
    def body(i, carry):
        sA, sC, sM = carry
        xv = xbuf[pl.ds(i * 16, 16)]
        yv = ybuf[pl.ds(i * 16, 16)]
        yf = yv.astype(jnp.float32)
        z = xv * (2.0 * yf - 1.0)
        zz = jnp.minimum(-z, 87.0)
        u = jnp.exp(zz)
        d = 1.0 + u
        di = lax.bitcast_convert_type(d, jnp.int32)
        e = lax.shift_right_arithmetic(di, 23) - 127
        m = lax.bitcast_convert_type(
            (di & 0x007FFFFF) | 0x3F800000, jnp.float32)
        f = m - 1.0
        poly = f * (_P0 + f * (_P1 + f * (_P2 + f * (_P3 + f * (_P4 + f * _P5)))))
        lnd = (e.astype(jnp.float32) + poly) * _LN2
        g = jnp.exp(1.5 * (zz - lnd))
        c = g * lnd
        return (sA + yf * c, sC + c, sM + yf)

    return lax.fori_loop(0, _SC_CHUNK // 16, body, carry_in)


def _sc_kernel_body(x_hbm, y_hbm, out_hbm,
                    xb0, xb1, yb0, yb1, acc, s0, s1, s2, s3):
    wid = lax.axis_index("s") * 2 + lax.axis_index("c")
    base = _N_TC + wid * (_SC_K * _SC_CHUNK)

    xbufs = (xb0, xb1)
    ybufs = (yb0, yb1)
    xsems = (s0, s1)
    ysems = (s2, s3)

    cps = []
    for k in range(_SC_K):
        b = k % 2
        off = base + k * _SC_CHUNK
        cpx = pltpu.make_async_copy(
            x_hbm.at[pl.ds(off, _SC_CHUNK)], xbufs[b], xsems[b])
        cpy = pltpu.make_async_copy(
            y_hbm.at[pl.ds(off, _SC_CHUNK)], ybufs[b], ysems[b])
        cpx.start()
        cpy.start()
        cps.append((cpx, cpy))

    zv = jnp.zeros((16,), jnp.float32)
    carry = (zv, zv, zv)
    for k in range(_SC_K):
        b = k % 2
        cps[k][0].wait()
        cps[k][1].wait()
        carry = _sc_chunk_sums(xbufs[b], ybufs[b], carry)
    sA, sC, sM = carry

    acc[0] = sA
    acc[1] = sC
    acc[2] = sM
    pltpu.sync_copy(acc, out_hbm.at[wid])


def _sc_call(xf, yf):
    mesh = plsc.VectorSubcoreMesh(core_axis_name="c", subcore_axis_name="s")
    fn = functools.partial(
        pl.kernel,
        out_type=jax.ShapeDtypeStruct((_SC_WORKERS, 3, 16), jnp.float32),
        mesh=mesh,
        scratch_types=[
            pltpu.VMEM((_SC_CHUNK,), jnp.float32),
            pltpu.VMEM((_SC_CHUNK,), jnp.float32),
            pltpu.VMEM((_SC_CHUNK,), jnp.int32),
            pltpu.VMEM((_SC_CHUNK,), jnp.int32),
            pltpu.VMEM((3, 16), jnp.float32),
            pltpu.SemaphoreType.DMA,
            pltpu.SemaphoreType.DMA,
            pltpu.SemaphoreType.DMA,
            pltpu.SemaphoreType.DMA,
        ],
    )(_sc_kernel_body)
    return fn(xf, yf)


def kernel(x, y):
    xf = x.reshape(_N)
    yflat = y.reshape(_N)
    x2 = xf.reshape(_ROWS, _COLS)
    y2 = yflat.reshape(_ROWS, _COLS)

    sc = _sc_call(xf, yflat)
    tc = _tc_call(x2, y2)

    A = tc[0] + jnp.sum(sc[:, 0, :])
    C = tc[1] + jnp.sum(sc[:, 1, :])
    M = tc[2] + jnp.sum(sc[:, 2, :])
    B = C - A
    loss = (_FACTOR * ((_N - M) / M) * A + (2.0 - _FACTOR) * B) / _N
    return loss


# speedup vs baseline: 2.0049x; 2.0049x over previous
import jax
import jax.numpy as jnp
from jax.experimental import pallas as pl
from jax.experimental.pallas import tpu as pltpu
import numpy as np

_SHAPE = (4, 1, 96, 128, 128)
_N = int(np.prod(_SHAPE))
_COLS = 128
_ROWS = _N // _COLS
_BLK_ROWS = 8192
_GRID = _ROWS // _BLK_ROWS

_FACTOR = 1.0
_GAMA = 1.5
_EPS = 1e-08


_CHUNK = 64
_NCHUNK = _BLK_ROWS // _CHUNK


def _body(x_ref, y_ref, out_ref, acc_ref):
    i = pl.program_id(0)

    def _step(k, carry):
        sA, sC, sM = carry
        xb = x_ref[pl.ds(k * _CHUNK, _CHUNK), :]
        yb = y_ref[pl.ds(k * _CHUNK, _CHUNK), :]
        yf = yb.astype(jnp.float32)
        z = xb * (2.0 * yf - 1.0)
        a = jnp.minimum(z * (-1.4426950408889634), 126.0)
        u = jnp.exp2(a)
        d = 1.0 + u
        L = jnp.log2(d)
        c = jnp.exp2(1.5 * (a - L)) * L
        return (sA + yf * c, sC + c, sM + yf)

    z0 = jnp.zeros((_CHUNK, _COLS), jnp.float32)
    carry = (z0, z0, z0)
    for k in range(_NCHUNK):
        carry = _step(k, carry)
    sA, sC, sM = carry

    @pl.when(i == 0)
    def _init():
        acc_ref[...] = jnp.zeros_like(acc_ref)

    acc_ref[0, :] += jnp.sum(sA, axis=0)
    acc_ref[1, :] += jnp.sum(sC, axis=0)
    acc_ref[2, :] += jnp.sum(sM, axis=0)

    @pl.when(i == _GRID - 1)
    def _fin():
        ln2 = 0.6931471805599453
        A = ln2 * jnp.sum(acc_ref[0, :])
        C = ln2 * jnp.sum(acc_ref[1, :])
        B = C - A
        M = jnp.sum(acc_ref[2, :])
        loss = (_FACTOR * ((_N - M) / M) * A + (2.0 - _FACTOR) * B) / _N
        out_ref[0] = loss


def kernel(x, y):
    x2 = x.reshape(_ROWS, _COLS)
    y2 = y.reshape(_ROWS, _COLS)
    out = pl.pallas_call(
        _body,
        grid=(_GRID,),
        in_specs=[
            pl.BlockSpec((_BLK_ROWS, _COLS), lambda i: (i, 0)),
            pl.BlockSpec((_BLK_ROWS, _COLS), lambda i: (i, 0)),
        ],
        out_specs=pl.BlockSpec(memory_space=pltpu.SMEM),
        out_shape=jax.ShapeDtypeStruct((1,), jnp.float32),
        scratch_shapes=[pltpu.VMEM((3, _COLS), jnp.float32)],
    )(x2, y2)
    return out[0]


# speedup vs baseline: 2.1150x; 1.0549x over previous
import jax
import jax.numpy as jnp
from jax.experimental import pallas as pl
from jax.experimental.pallas import tpu as pltpu
import numpy as np

_SHAPE = (4, 1, 96, 128, 128)
_N = int(np.prod(_SHAPE))
_COLS = 128
_ROWS = _N // _COLS
_BLK_ROWS = 8192
_GRID = _ROWS // _BLK_ROWS

_FACTOR = 1.0
_GAMA = 1.5
_EPS = 1e-08


_CHUNK = 32
_NCHUNK = _BLK_ROWS // _CHUNK


def _body(x_ref, y_ref, out_ref, acc_ref):
    i = pl.program_id(0)

    def _step(k, carry):
        sW, sC, sMi = carry
        xb = x_ref[pl.ds(k * _CHUNK, _CHUNK), :]
        yb = y_ref[pl.ds(k * _CHUNK, _CHUNK), :]
        w = jnp.where(yb > 0, -1.4426950408889634, 1.4426950408889634)
        a = jnp.minimum(xb * w, 126.0)
        u = jnp.exp2(a)
        d = 1.0 + u
        L = jnp.log2(d)
        c = jnp.exp2(1.5 * (a - L)) * L
        return (sW + w * c, sC + c, sMi + yb)

    z0 = jnp.zeros((_CHUNK, _COLS), jnp.float32)
    zi = jnp.zeros((_CHUNK, _COLS), jnp.int32)
    carry = (z0, z0, zi)
    for k in range(_NCHUNK):
        carry = _step(k, carry)
    sW, sC, sMi = carry

    @pl.when(i == 0)
    def _init():
        acc_ref[...] = jnp.zeros_like(acc_ref)

    acc_ref[0, :] += jnp.sum(sW, axis=0)
    acc_ref[1, :] += jnp.sum(sC, axis=0)
    acc_ref[2, :] += jnp.sum(sMi, axis=0).astype(jnp.float32)

    @pl.when(i == _GRID - 1)
    def _fin():
        ln2 = 0.6931471805599453
        C = ln2 * jnp.sum(acc_ref[1, :])
        A = 0.5 * (C - ln2 * ln2 * jnp.sum(acc_ref[0, :]))
        B = C - A
        M = jnp.sum(acc_ref[2, :])
        loss = (_FACTOR * ((_N - M) / M) * A + (2.0 - _FACTOR) * B) / _N
        out_ref[0] = loss


def kernel(x, y):
    x2 = x.reshape(_ROWS, _COLS)
    y2 = y.reshape(_ROWS, _COLS)
    out = pl.pallas_call(
        _body,
        grid=(_GRID,),
        in_specs=[
            pl.BlockSpec((_BLK_ROWS, _COLS), lambda i: (i, 0)),
            pl.BlockSpec((_BLK_ROWS, _COLS), lambda i: (i, 0)),
        ],
        out_specs=pl.BlockSpec(memory_space=pltpu.SMEM),
        out_shape=jax.ShapeDtypeStruct((1,), jnp.float32),
        scratch_shapes=[pltpu.VMEM((3, _COLS), jnp.float32)],
    )(x2, y2)
    return out[0]
